# baseline (speedup 1.0000x reference)
import functools

import jax
import jax.numpy as jnp
from jax import lax
from jax.experimental import pallas as pl
from jax.experimental.pallas import tpu as pltpu
from jax.experimental.pallas import tpu_sc as plsc

L = 16
NC = 2
NS = 16

FIX_CAP = 256
FIXCH = 32


def _take16(x, idx):
    dnums = lax.GatherDimensionNumbers(
        offset_dims=(), collapsed_slice_dims=(0,), start_index_map=(0,))
    return lax.gather(x, idx[:, None], dnums, slice_sizes=(1,),
                      mode=lax.GatherScatterMode.PROMISE_IN_BOUNDS)


def _build(n_nodes, batch, dim):
    nr = -(-n_nodes // NS)
    nr_pad = -(-nr // (4 * L)) * (4 * L)
    half = batch // NC
    rpt = half // NS
    mesh = plsc.VectorSubcoreMesh(core_axis_name="c", subcore_axis_name="s")

    @functools.partial(
        pl.kernel,
        out_type=jax.ShapeDtypeStruct((batch, dim), jnp.float32),
        mesh=mesh,
        compiler_params=pltpu.CompilerParams(needs_layout_passes=False),
        scratch_types=[
            pltpu.VMEM((batch + 2 * L,), jnp.int32),
            pltpu.VMEM((nr_pad,), jnp.int32),
            pltpu.VMEM((FIX_CAP + L,), jnp.int32),
            pltpu.VMEM((FIX_CAP + L,), jnp.int32),
            pltpu.VMEM((FIX_CAP // FIXCH, FIXCH), jnp.int32),
            pltpu.VMEM((FIXCH, dim), jnp.float32),
            pltpu.VMEM((2, 128, dim), jnp.float32),
            pltpu.SemaphoreType.DMA,
            pltpu.SemaphoreType.DMA,
        ],
    )
    def sc_kernel(idx_hbm, values_hbm, out_hbm,
                  idx_v, table_v, fix_k, fix_w, fix_k2,
                  rows_v, stage_v, sem, sem2):
        c = lax.axis_index("c")
        s = lax.axis_index("s")
        base = s * nr
        half_lo = c * half
        lane = lax.iota(jnp.int32, L)
        zero16 = jnp.zeros((L,), jnp.int32)
        nxt_perm = jnp.minimum(lane + 1, L - 1)

        k_lo = half_lo + s * rpt

        c1_in = pltpu.async_copy(values_hbm.at[pl.ds(k_lo, 128)],
                                 stage_v.at[0], sem2)
        copy_idx = pltpu.async_copy(idx_hbm, idx_v.at[pl.ds(0, batch)], sem)

        copy_idx.wait()

        def sort_stage(t):
            v_idx = idx_v[pl.ds(t * L, L)]
            skey, _ = plsc.sort_key_val((v_idx << 4) | lane, lane)
            id_s = skey >> 4
            j_vec = t * L + (skey & (L - 1))
            nxt = _take16(skey, nxt_perm)
            last_run = (id_s != (nxt >> 4)) | (lane == L - 1)
            off = id_s - base
            in_range = (off >= 0) & (off < nr)
            offc = jnp.clip(off, 0, nr - 1)
            return offc, j_vec, last_run & in_range

        def pass_a1(i, carry):
            st_a, st_b = carry
            nxt_a = sort_stage(2 * i + 2)
            offc, j_vec, do_st = st_a
            plsc.store_scatter(table_v, [offc], j_vec + 1, mask=do_st)
            nxt_b = sort_stage(2 * i + 3)
            offc, j_vec, do_st = st_b
            plsc.store_scatter(table_v, [offc], j_vec + 1, mask=do_st)
            return nxt_a, nxt_b

        lax.fori_loop(0, batch // (2 * L), pass_a1,
                      (sort_stage(0), sort_stage(1)))

        t0 = c * (half // L)

        def load_stage(t):
            v_idx = idx_v[pl.ds(t * L, L)]
            off = v_idx - base
            in_range = (off >= 0) & (off < nr)
            offc = jnp.clip(off, 0, nr - 1)
            return plsc.load_gather(table_v, [offc]), in_range, t * L + lane

        def a2_step(m, st):
            w1, in_range, j_vec = st
            wv = w1 - 1
            has_loser = in_range & (wv != j_vec)
            plsc.store_compressed(fix_k.at[pl.ds(m, L)], j_vec,
                                  mask=has_loser)
            plsc.store_compressed(fix_w.at[pl.ds(m, L)], wv,
                                  mask=has_loser)
            return m + plsc.all_reduce_population_count(has_loser)[0]

        def pass_a2(i, carry):
            m, st_a, st_b = carry
            nxt_a = load_stage(t0 + 2 * i + 2)
            m = a2_step(m, st_a)
            nxt_b = load_stage(t0 + 2 * i + 3)
            m = a2_step(m, st_b)
            return m, nxt_a, nxt_b

        m = lax.fori_loop(0, half // (2 * L), pass_a2,
                          (jnp.int32(0), load_stage(t0),
                           load_stage(t0 + 1)))[0]

        def pass_b(i, carry):
            k0, w0 = carry
            kv = fix_k[pl.ds(i * L, L)]
            wv = fix_w[pl.ds(i * L, L)]
            if_0 = i == 0
            k0 = jnp.where(if_0, _take16(kv, zero16), k0)
            w0 = jnp.where(if_0, _take16(wv, zero16), w0)
            valid = (i * L + lane) < m
            kv = jnp.where(valid, kv, k0)
            wv = jnp.where(valid, wv, w0)
            fix_w[pl.ds(i * L, L)] = wv
            r = i // (FIXCH // L)
            fix_k2.at[r][pl.ds((i % (FIXCH // L)) * L, L)] = kv
            return (k0, w0)
        lax.fori_loop(0, FIX_CAP // L, pass_b, (zero16, zero16))

        ncc = rpt // 128
        g = c1_in
        for i in range(ncc):
            g.wait()
            if i + 1 < ncc:
                g = pltpu.async_copy(
                    values_hbm.at[pl.ds(k_lo + (i + 1) * 128, 128)],
                    stage_v.at[(i + 1) % 2], sem2)
            pltpu.async_copy(stage_v.at[i % 2],
                             out_hbm.at[pl.ds(k_lo + i * 128, 128)],
                             sem).wait()
        plsc.subcore_barrier()

        def pass_c(ci, _):
            @pl.when(ci * FIXCH < m)
            def _():
                w_view = fix_w.at[pl.ds(ci * FIXCH, FIXCH)]
                pltpu.async_copy(values_hbm.at[w_view], rows_v, sem).wait()
                pltpu.async_copy(rows_v, out_hbm.at[fix_k2.at[ci]], sem).wait()
            return 0
        lax.fori_loop(0, FIX_CAP // FIXCH, pass_c, 0)

    return sc_kernel


def kernel(memory, node_idxs, values):
    n_nodes, dim = memory.shape
    batch = node_idxs.shape[0]
    sc_kernel = _build(n_nodes, batch, dim)
    return sc_kernel(node_idxs.astype(jnp.int32), values)


# speedup vs baseline: 1.0374x; 1.0374x over previous
import functools

import jax
import jax.numpy as jnp
from jax import lax
from jax.experimental import pallas as pl
from jax.experimental.pallas import tpu as pltpu
from jax.experimental.pallas import tpu_sc as plsc

L = 16
NC = 2
NS = 16

FIX_CAP = 256
FIXCH = 32


def _take16(x, idx):
    dnums = lax.GatherDimensionNumbers(
        offset_dims=(), collapsed_slice_dims=(0,), start_index_map=(0,))
    return lax.gather(x, idx[:, None], dnums, slice_sizes=(1,),
                      mode=lax.GatherScatterMode.PROMISE_IN_BOUNDS)


def _build(n_nodes, batch, dim):
    nr = -(-n_nodes // NS)
    nr_pad = -(-nr // (4 * L)) * (4 * L)
    half = batch // NC
    rpt = half // NS
    mesh = plsc.VectorSubcoreMesh(core_axis_name="c", subcore_axis_name="s")

    @functools.partial(
        pl.kernel,
        out_type=jax.ShapeDtypeStruct((batch, dim), jnp.float32),
        mesh=mesh,
        compiler_params=pltpu.CompilerParams(needs_layout_passes=False),
        scratch_types=[
            pltpu.VMEM((batch + 2 * L,), jnp.int32),
            pltpu.VMEM((nr_pad,), jnp.int32),
            pltpu.VMEM((FIX_CAP + L,), jnp.int32),
            pltpu.VMEM((FIX_CAP + L,), jnp.int32),
            pltpu.VMEM((FIX_CAP // FIXCH, FIXCH), jnp.int32),
            pltpu.VMEM((FIXCH, dim), jnp.float32),
            pltpu.VMEM((2, 128, dim), jnp.float32),
            pltpu.SemaphoreType.DMA,
            pltpu.SemaphoreType.DMA,
            pltpu.SemaphoreType.DMA,
            pltpu.SemaphoreType.DMA,
            pltpu.SemaphoreType.DMA,
            pltpu.SemaphoreType.DMA,
        ],
    )
    def sc_kernel(idx_hbm, values_hbm, out_hbm,
                  idx_v, table_v, fix_k, fix_w, fix_k2,
                  rows_v, stage_v, sem, semi0, semi1,
                  semo0, semo1, sem3):
        c = lax.axis_index("c")
        s = lax.axis_index("s")
        base = s * nr
        half_lo = c * half
        lane = lax.iota(jnp.int32, L)
        zero16 = jnp.zeros((L,), jnp.int32)
        nxt_perm = jnp.minimum(lane + 1, L - 1)

        k_lo = half_lo + s * rpt

        def c1_in(i, b, s_):
            return pltpu.async_copy(
                values_hbm.at[pl.ds(k_lo + i * 128, 128)], stage_v.at[b], s_)

        copy_idx = pltpu.async_copy(idx_hbm, idx_v.at[pl.ds(0, batch)], sem)
        ncc = rpt // 128
        c1_ins = [c1_in(0, 0, semi0), c1_in(1, 1, semi1)]

        copy_idx.wait()

        def sort_stage(t):
            v_idx = idx_v[pl.ds(t * L, L)]
            skey, _ = plsc.sort_key_val((v_idx << 4) | lane, lane)
            id_s = skey >> 4
            j_vec = t * L + (skey & (L - 1))
            nxt = _take16(skey, nxt_perm)
            last_run = (id_s != (nxt >> 4)) | (lane == L - 1)
            off = id_s - base
            in_range = (off >= 0) & (off < nr)
            offc = jnp.clip(off, 0, nr - 1)
            return offc, j_vec, last_run & in_range

        def pass_a1(i, carry):
            st_a, st_b = carry
            nxt_a = sort_stage(2 * i + 2)
            offc, j_vec, do_st = st_a
            plsc.store_scatter(table_v, [offc], j_vec + 1, mask=do_st)
            nxt_b = sort_stage(2 * i + 3)
            offc, j_vec, do_st = st_b
            plsc.store_scatter(table_v, [offc], j_vec + 1, mask=do_st)
            return nxt_a, nxt_b

        lax.fori_loop(0, batch // (2 * L), pass_a1,
                      (sort_stage(0), sort_stage(1)))

        t0 = c * (half // L)

        def load_stage(t):
            v_idx = idx_v[pl.ds(t * L, L)]
            off = v_idx - base
            in_range = (off >= 0) & (off < nr)
            offc = jnp.clip(off, 0, nr - 1)
            return plsc.load_gather(table_v, [offc]), in_range, t * L + lane

        def pass_a2(i, carry):
            m, w1, in_range, j_vec = carry
            nxt_st = load_stage(t0 + i + 1)
            wv = w1 - 1
            has_loser = in_range & (wv != j_vec)
            plsc.store_compressed(fix_k.at[pl.ds(m, L)], j_vec,
                                  mask=has_loser)
            plsc.store_compressed(fix_w.at[pl.ds(m, L)], wv,
                                  mask=has_loser)
            m = m + plsc.all_reduce_population_count(has_loser)[0]
            return (m,) + nxt_st

        m = lax.fori_loop(0, half // L, pass_a2,
                          (jnp.int32(0),) + load_stage(t0))[0]

        def pass_b(i, carry):
            k0, w0 = carry
            kv = fix_k[pl.ds(i * L, L)]
            wv = fix_w[pl.ds(i * L, L)]
            if_0 = i == 0
            k0 = jnp.where(if_0, _take16(kv, zero16), k0)
            w0 = jnp.where(if_0, _take16(wv, zero16), w0)
            valid = (i * L + lane) < m
            kv = jnp.where(valid, kv, k0)
            wv = jnp.clip(jnp.where(valid, wv, w0), 0, batch - 1)
            fix_w[pl.ds(i * L, L)] = wv
            r = i // (FIXCH // L)
            fix_k2.at[r][pl.ds((i % (FIXCH // L)) * L, L)] = kv
            return (k0, w0)
        lax.fori_loop(0, FIX_CAP // L, pass_b, (zero16, zero16))

        g2 = pltpu.async_copy(values_hbm.at[fix_w.at[pl.ds(0, FIXCH)]],
                              rows_v, sem3)

        for p in range(ncc // 2):
            a = 2 * p
            c1_ins[0].wait()
            out_a = pltpu.async_copy(
                stage_v.at[0], out_hbm.at[pl.ds(k_lo + a * 128, 128)], semo0)
            c1_ins[1].wait()
            out_b = pltpu.async_copy(
                stage_v.at[1],
                out_hbm.at[pl.ds(k_lo + (a + 1) * 128, 128)], semo1)
            out_a.wait()
            out_b.wait()
            if p + 1 < ncc // 2:
                c1_ins = [c1_in(a + 2, 0, semi0), c1_in(a + 3, 1, semi1)]
        g2.wait()
        plsc.subcore_barrier()

        @pl.when(m > 0)
        def _():
            pltpu.async_copy(rows_v, out_hbm.at[fix_k2.at[0]], sem).wait()

        def pass_c(ci, _):
            @pl.when(ci * FIXCH < m)
            def _():
                w_view = fix_w.at[pl.ds(ci * FIXCH, FIXCH)]
                pltpu.async_copy(values_hbm.at[w_view], rows_v, sem).wait()
                pltpu.async_copy(rows_v, out_hbm.at[fix_k2.at[ci]], sem).wait()
            return 0
        lax.fori_loop(1, FIX_CAP // FIXCH, pass_c, 0)

    return sc_kernel


def kernel(memory, node_idxs, values):
    n_nodes, dim = memory.shape
    batch = node_idxs.shape[0]
    sc_kernel = _build(n_nodes, batch, dim)
    return sc_kernel(node_idxs.astype(jnp.int32), values)
